# Initial kernel scaffold; baseline (speedup 1.0000x reference)
import jax
import jax.numpy as jnp
from jax.experimental import pallas as pl


def kernel(query, relation_emb, indicator, Wr0, br0, Wl0, bl0, Wr1, br1, Wl1, bl1, W1, b1, W2, b2):
    raise NotImplementedError("write your pallas kernel here")


# speedup vs baseline: 57.3851x; 57.3851x over previous
import functools

import jax
import jax.numpy as jnp
from jax.experimental import pallas as pl
from jax.experimental.pallas import tpu as pltpu

D = 128
R2 = 16
B = 64
EPS = 1e-6
F32 = jnp.float32


def _features(m):
    mean = m * 0.5
    mx = jnp.maximum(m, 0.0)
    mn = jnp.minimum(m, 0.0)
    var = (m * m) * 0.5 - mean * mean
    std = jnp.sqrt(jnp.maximum(var, EPS))
    return jnp.concatenate([mean, mx, mn, std], axis=1)


def _rule_kernel(q_ref, wr0_ref, br0_ref, wr1_ref, br1_ref, c_ref,
                 wf0_ref, bl0_ref, wf1_ref, beff1_ref,
                 w1a_ref, w1b_ref, b1_ref, w2_ref, b2_ref, emb_ref,
                 sub0_ref, sub1_ref,
                 hid1_s, rel1_s, qw1_s, score_s):
    i = pl.program_id(0)

    @pl.when(i == 0)
    def _layer0():
        q = q_ref[...]
        rel1_s[...] = (
            jnp.dot(q, wr1_ref[...], preferred_element_type=F32) + br1_ref[...]
        )
        qw1_s[...] = (
            jnp.dot(q, w1b_ref[...], preferred_element_type=F32) + b1_ref[...]
        )
        rel0 = jnp.dot(q, wr0_ref[...], preferred_element_type=F32) + br0_ref[...]
        c = c_ref[...]
        wf0 = wf0_ref[...]
        bl0 = bl0_ref[...]
        for r in range(R2):
            m = c * rel0[:, r * D:(r + 1) * D]
            f = _features(m)
            hid1_s[r] = jnp.maximum(
                jnp.dot(f, wf0, preferred_element_type=F32) + bl0, 0.0)

    h1 = hid1_s[i]
    rel1 = rel1_s[...]
    qw1 = qw1_s[...]
    wf1 = wf1_ref[...]
    beff1 = beff1_ref[...]
    w1a = w1a_ref[...]
    w2 = w2_ref[...]
    b2 = b2_ref[...]
    cols = []
    for r1 in range(R2):
        m2 = h1 * rel1[:, r1 * D:(r1 + 1) * D]
        f2 = _features(m2)
        hid2 = jnp.maximum(
            jnp.dot(f2, wf1, preferred_element_type=F32) + beff1, 0.0)
        ho = jnp.maximum(
            jnp.dot(hid2, w1a, preferred_element_type=F32) + qw1, 0.0)
        cols.append(jnp.dot(ho, w2, preferred_element_type=F32) + b2)
    score_s[i] = jnp.concatenate(cols, axis=1)

    @pl.when(i == R2 - 1)
    def _finish():
        s_all = score_s[...]
        mx = jnp.max(jnp.max(s_all, axis=0), axis=1)[None, :, None]
        e = jnp.exp(s_all - mx)
        den = jnp.sum(jnp.sum(e, axis=0), axis=1)[None, :, None]
        att = e / den
        marg0 = jnp.sum(att, axis=2)
        marg1 = jnp.sum(att, axis=0)
        emb = emb_ref[...]
        sub0_ref[...] = jax.lax.dot_general(
            marg0, emb, (((0,), (0,)), ((), ())), preferred_element_type=F32)
        sub1_ref[...] = jnp.dot(marg1, emb, preferred_element_type=F32)


@functools.partial(jax.jit, static_argnames=("interpret",))
def _run(query, relation_emb, indicator, Wr0, br0, Wl0, bl0,
         Wr1, br1, Wl1, bl1, W1, b1, W2, b2, interpret=False):
    scales = jnp.array([1.0, 1.5, 2.0 / 3.0], dtype=F32)
    wl0r = Wl0[D:].reshape(D, 4, 3, D)
    wf0 = jnp.einsum("dksD,s->kdD", wl0r, scales).reshape(4 * D, D)
    wl1r = Wl1[D:].reshape(D, 4, 3, D)
    wf1 = jnp.einsum("dksD,s->kdD", wl1r, scales).reshape(4 * D, D)
    stdc = jnp.sqrt(jnp.asarray(EPS, F32))
    h2l0 = jax.nn.relu(stdc * jnp.einsum("dsD,s->D", wl0r[:, 3], scales) + bl0)
    beff1 = h2l0 @ Wl1[:D] + bl1

    spec = lambda shape: pl.BlockSpec(shape, lambda i: tuple(0 for _ in shape))
    out = pl.pallas_call(
        _rule_kernel,
        grid=(R2,),
        in_specs=[
            spec((B, D)),
            spec((D, R2 * D)),
            spec((1, R2 * D)),
            spec((D, R2 * D)),
            spec((1, R2 * D)),
            spec((1, D)),
            spec((4 * D, D)),
            spec((1, D)),
            spec((4 * D, D)),
            spec((1, D)),
            spec((D, D)),
            spec((D, D)),
            spec((1, D)),
            spec((D, 1)),
            spec((1, 1)),
            spec((R2, D)),
        ],
        out_specs=[spec((B, D)), spec((B, D))],
        out_shape=[
            jax.ShapeDtypeStruct((B, D), F32),
            jax.ShapeDtypeStruct((B, D), F32),
        ],
        scratch_shapes=[
            pltpu.VMEM((R2, B, D), F32),
            pltpu.VMEM((B, R2 * D), F32),
            pltpu.VMEM((B, D), F32),
            pltpu.VMEM((R2, B, R2), F32),
        ],
        interpret=interpret,
    )(query, Wr0, br0.reshape(1, R2 * D), Wr1, br1.reshape(1, R2 * D),
      indicator, wf0, bl0.reshape(1, D), wf1, beff1.reshape(1, D),
      W1[:D], W1[D:], b1.reshape(1, D), W2, b2.reshape(1, 1), relation_emb)
    sub0, sub1 = out
    return jnp.stack([sub0, sub1], axis=1)


def kernel(query, relation_emb, indicator, Wr0, br0, Wl0, bl0,
           Wr1, br1, Wl1, bl1, W1, b1, W2, b2):
    subgoals = _run(query, relation_emb, indicator, Wr0, br0, Wl0, bl0,
                    Wr1, br1, Wl1, bl1, W1, b1, W2, b2)
    masks = jnp.ones(subgoals.shape[:-1], dtype=bool)
    return (subgoals, masks)


# speedup vs baseline: 147.7344x; 2.5744x over previous
import functools

import jax
import jax.numpy as jnp
from jax.experimental import pallas as pl
from jax.experimental.pallas import tpu as pltpu

D = 128
R2 = 16
B = 64
EPS = 1e-6
F32 = jnp.float32


def _features(m):
    mean = m * 0.5
    mx = jnp.maximum(m, 0.0)
    mn = jnp.minimum(m, 0.0)
    var = (m * m) * 0.5 - mean * mean
    std = jnp.sqrt(jnp.maximum(var, EPS))
    return jnp.concatenate([mean, mx, mn, std], axis=1)


def _rule_kernel(q_ref, wr0_ref, br0_ref, wr1_ref, br1_ref, c_ref,
                 wf0_ref, bl0_ref, wf1_ref, beff1_ref,
                 w1a_ref, w1b_ref, b1_ref, w2_ref, b2_ref, emb_ref,
                 sub0_ref, sub1_ref,
                 hid1_s, rel1_s, qw1_s, score_s):
    i = pl.program_id(0)

    @pl.when(i == 0)
    def _layer0():
        q = q_ref[...]
        rel1 = jnp.dot(q, wr1_ref[...], preferred_element_type=F32) + br1_ref[...]
        rel1_s[...] = jnp.concatenate(
            [rel1[:, r * D:(r + 1) * D] for r in range(R2)], axis=0)
        qw1 = jnp.dot(q, w1b_ref[...], preferred_element_type=F32) + b1_ref[...]
        qw1_s[...] = jnp.concatenate([qw1] * R2, axis=0)
        rel0 = jnp.dot(q, wr0_ref[...], preferred_element_type=F32) + br0_ref[...]
        m1 = c_ref[...] * jnp.concatenate(
            [rel0[:, r * D:(r + 1) * D] for r in range(R2)], axis=0)
        f1 = _features(m1)
        hid1_s[...] = jnp.maximum(
            jnp.dot(f1, wf0_ref[...], preferred_element_type=F32)
            + bl0_ref[...], 0.0)

    h1 = hid1_s[pl.ds(i * B, B), :]
    m2 = jnp.concatenate([h1] * R2, axis=0) * rel1_s[...]
    f2 = _features(m2)
    hid2 = jnp.maximum(
        jnp.dot(f2, wf1_ref[...], preferred_element_type=F32)
        + beff1_ref[...], 0.0)
    ho = jnp.maximum(
        jnp.dot(hid2, w1a_ref[...], preferred_element_type=F32)
        + qw1_s[...], 0.0)
    sc = jnp.dot(ho, w2_ref[...], preferred_element_type=F32) + b2_ref[...]
    score_s[i] = jnp.concatenate(
        [sc[r * B:(r + 1) * B, :] for r in range(R2)], axis=1)

    @pl.when(i == R2 - 1)
    def _finish():
        s_all = score_s[...]
        mx = jnp.max(jnp.max(s_all, axis=0), axis=1)[None, :, None]
        e = jnp.exp(s_all - mx)
        den = jnp.sum(jnp.sum(e, axis=0), axis=1)[None, :, None]
        att = e / den
        marg0 = jnp.sum(att, axis=2)
        marg1 = jnp.sum(att, axis=0)
        emb = emb_ref[...]
        sub0_ref[...] = jax.lax.dot_general(
            marg0, emb, (((0,), (0,)), ((), ())), preferred_element_type=F32)
        sub1_ref[...] = jnp.dot(marg1, emb, preferred_element_type=F32)


@functools.partial(jax.jit, static_argnames=("interpret",))
def _run(query, relation_emb, indicator, Wr0, br0, Wl0, bl0,
         Wr1, br1, Wl1, bl1, W1, b1, W2, b2, interpret=False):
    scales = jnp.array([1.0, 1.5, 2.0 / 3.0], dtype=F32)
    wl0r = Wl0[D:].reshape(D, 4, 3, D)
    wf0 = jnp.einsum("dksD,s->kdD", wl0r, scales).reshape(4 * D, D)
    wl1r = Wl1[D:].reshape(D, 4, 3, D)
    wf1 = jnp.einsum("dksD,s->kdD", wl1r, scales).reshape(4 * D, D)
    stdc = jnp.sqrt(jnp.asarray(EPS, F32))
    h2l0 = jax.nn.relu(stdc * jnp.einsum("dsD,s->D", wl0r[:, 3], scales) + bl0)
    beff1 = h2l0 @ Wl1[:D] + bl1

    spec = lambda shape: pl.BlockSpec(shape, lambda i: tuple(0 for _ in shape))
    out = pl.pallas_call(
        _rule_kernel,
        grid=(R2,),
        in_specs=[
            spec((B, D)),
            spec((D, R2 * D)),
            spec((1, R2 * D)),
            spec((D, R2 * D)),
            spec((1, R2 * D)),
            spec((1, D)),
            spec((4 * D, D)),
            spec((1, D)),
            spec((4 * D, D)),
            spec((1, D)),
            spec((D, D)),
            spec((D, D)),
            spec((1, D)),
            spec((D, 1)),
            spec((1, 1)),
            spec((R2, D)),
        ],
        out_specs=[spec((B, D)), spec((B, D))],
        out_shape=[
            jax.ShapeDtypeStruct((B, D), F32),
            jax.ShapeDtypeStruct((B, D), F32),
        ],
        scratch_shapes=[
            pltpu.VMEM((R2 * B, D), F32),
            pltpu.VMEM((R2 * B, D), F32),
            pltpu.VMEM((R2 * B, D), F32),
            pltpu.VMEM((R2, B, R2), F32),
        ],
        interpret=interpret,
    )(query, Wr0, br0.reshape(1, R2 * D), Wr1, br1.reshape(1, R2 * D),
      indicator, wf0, bl0.reshape(1, D), wf1, beff1.reshape(1, D),
      W1[:D], W1[D:], b1.reshape(1, D), W2, b2.reshape(1, 1), relation_emb)
    sub0, sub1 = out
    return jnp.stack([sub0, sub1], axis=1)


def kernel(query, relation_emb, indicator, Wr0, br0, Wl0, bl0,
           Wr1, br1, Wl1, bl1, W1, b1, W2, b2):
    subgoals = _run(query, relation_emb, indicator, Wr0, br0, Wl0, bl0,
                    Wr1, br1, Wl1, bl1, W1, b1, W2, b2)
    masks = jnp.ones(subgoals.shape[:-1], dtype=bool)
    return (subgoals, masks)


# speedup vs baseline: 148.5475x; 1.0055x over previous
import functools

import jax
import jax.numpy as jnp
from jax.experimental import pallas as pl
from jax.experimental.pallas import tpu as pltpu

D = 128
R2 = 16
B = 64
EPS = 1e-6
F32 = jnp.float32


_STDC = 0.0010000000474974513


def _features(m):
    mean = m * 0.5
    mx = jnp.maximum(m, 0.0)
    mn = jnp.minimum(m, 0.0)
    std = jnp.maximum(jnp.abs(mean), _STDC)
    return jnp.concatenate([mean, mx, mn, std], axis=1)


def _rule_kernel(q_ref, wr0_ref, br0_ref, wr1_ref, br1_ref, c_ref,
                 wf0_ref, bl0_ref, wf1_ref, beff1_ref,
                 w1a_ref, w1b_ref, b1_ref, w2_ref, b2_ref, emb_ref,
                 sub0_ref, sub1_ref,
                 hid1_s, rel1_s, qw1_s, score_s):
    i = pl.program_id(0)

    @pl.when(i == 0)
    def _layer0():
        q = q_ref[...]
        c = c_ref[...]
        for r in range(R2):
            lo, hi = r * D, (r + 1) * D
            rel1_s[r * B:(r + 1) * B, :] = (
                jnp.dot(q, wr1_ref[:, lo:hi], preferred_element_type=F32)
                + br1_ref[:, lo:hi])
            hid1_s[r * B:(r + 1) * B, :] = c * (
                jnp.dot(q, wr0_ref[:, lo:hi], preferred_element_type=F32)
                + br0_ref[:, lo:hi])
        qw1 = jnp.dot(q, w1b_ref[...], preferred_element_type=F32) + b1_ref[...]
        qw1_s[...] = jnp.broadcast_to(qw1[None], (R2, B, D)).reshape(R2 * B, D)
        f1 = _features(hid1_s[...])
        hid1_s[...] = jnp.maximum(
            jnp.dot(f1, wf0_ref[...], preferred_element_type=F32)
            + bl0_ref[...], 0.0)

    h1 = hid1_s[pl.ds(i * B, B), :]
    m2 = (jnp.broadcast_to(h1[None], (R2, B, D)).reshape(R2 * B, D)
          * rel1_s[...])
    f2 = _features(m2)
    hid2 = jnp.maximum(
        jnp.dot(f2, wf1_ref[...], preferred_element_type=F32)
        + beff1_ref[...], 0.0)
    ho = jnp.maximum(
        jnp.dot(hid2, w1a_ref[...], preferred_element_type=F32)
        + qw1_s[...], 0.0)
    sc = jnp.dot(ho, w2_ref[...], preferred_element_type=F32) + b2_ref[...]
    score_s[i] = jnp.concatenate(
        [sc[r * B:(r + 1) * B, :] for r in range(R2)], axis=1)

    @pl.when(i == R2 - 1)
    def _finish():
        s_all = score_s[...]
        mx = jnp.max(jnp.max(s_all, axis=0), axis=1)[None, :, None]
        e = jnp.exp(s_all - mx)
        den = jnp.sum(jnp.sum(e, axis=0), axis=1)[None, :, None]
        att = e / den
        marg0 = jnp.sum(att, axis=2)
        marg1 = jnp.sum(att, axis=0)
        emb = emb_ref[...]
        sub0_ref[...] = jax.lax.dot_general(
            marg0, emb, (((0,), (0,)), ((), ())), preferred_element_type=F32)
        sub1_ref[...] = jnp.dot(marg1, emb, preferred_element_type=F32)


@functools.partial(jax.jit, static_argnames=("interpret",))
def _run(query, relation_emb, indicator, Wr0, br0, Wl0, bl0,
         Wr1, br1, Wl1, bl1, W1, b1, W2, b2, interpret=False):
    scales = jnp.array([1.0, 1.5, 2.0 / 3.0], dtype=F32)
    wl0r = Wl0[D:].reshape(D, 4, 3, D)
    wf0 = jnp.einsum("dksD,s->kdD", wl0r, scales).reshape(4 * D, D)
    wl1r = Wl1[D:].reshape(D, 4, 3, D)
    wf1 = jnp.einsum("dksD,s->kdD", wl1r, scales).reshape(4 * D, D)
    stdc = jnp.sqrt(jnp.asarray(EPS, F32))
    h2l0 = jax.nn.relu(stdc * jnp.einsum("dsD,s->D", wl0r[:, 3], scales) + bl0)
    beff1 = h2l0 @ Wl1[:D] + bl1

    spec = lambda shape: pl.BlockSpec(shape, lambda i: tuple(0 for _ in shape))
    out = pl.pallas_call(
        _rule_kernel,
        grid=(R2,),
        in_specs=[
            spec((B, D)),
            spec((D, R2 * D)),
            spec((1, R2 * D)),
            spec((D, R2 * D)),
            spec((1, R2 * D)),
            spec((1, D)),
            spec((4 * D, D)),
            spec((1, D)),
            spec((4 * D, D)),
            spec((1, D)),
            spec((D, D)),
            spec((D, D)),
            spec((1, D)),
            spec((D, 1)),
            spec((1, 1)),
            spec((R2, D)),
        ],
        out_specs=[spec((B, D)), spec((B, D))],
        out_shape=[
            jax.ShapeDtypeStruct((B, D), F32),
            jax.ShapeDtypeStruct((B, D), F32),
        ],
        scratch_shapes=[
            pltpu.VMEM((R2 * B, D), F32),
            pltpu.VMEM((R2 * B, D), F32),
            pltpu.VMEM((R2 * B, D), F32),
            pltpu.VMEM((R2, B, R2), F32),
        ],
        interpret=interpret,
    )(query, Wr0, br0.reshape(1, R2 * D), Wr1, br1.reshape(1, R2 * D),
      indicator, wf0, bl0.reshape(1, D), wf1, beff1.reshape(1, D),
      W1[:D], W1[D:], b1.reshape(1, D), W2, b2.reshape(1, 1), relation_emb)
    sub0, sub1 = out
    return jnp.stack([sub0, sub1], axis=1)


def kernel(query, relation_emb, indicator, Wr0, br0, Wl0, bl0,
           Wr1, br1, Wl1, bl1, W1, b1, W2, b2):
    subgoals = _run(query, relation_emb, indicator, Wr0, br0, Wl0, bl0,
                    Wr1, br1, Wl1, bl1, W1, b1, W2, b2)
    masks = jnp.ones(subgoals.shape[:-1], dtype=bool)
    return (subgoals, masks)


# speedup vs baseline: 189.0097x; 1.2724x over previous
import functools

import jax
import jax.numpy as jnp
from jax.experimental import pallas as pl
from jax.experimental.pallas import tpu as pltpu

D = 128
R2 = 16
B = 64
F32 = jnp.float32
_STDC = 0.0010000000474974513
_SCALES = (1.0, 1.5, 2.0 / 3.0)


def _features(m):
    mean = m * 0.5
    mx = jnp.maximum(m, 0.0)
    mn = jnp.minimum(m, 0.0)
    std = jnp.maximum(jnp.abs(mean), _STDC)
    return jnp.concatenate([mean, mx, mn, std], axis=1)


def _fold(wl):
    x = wl[D:, :].reshape(D, 12, D)
    blocks = []
    for k in range(4):
        acc = None
        for s, sc in enumerate(_SCALES):
            sl = x[:, k * 3 + s, :]
            term = sl * sc if sc != 1.0 else sl
            acc = term if acc is None else acc + term
        blocks.append(acc)
    return jnp.concatenate(blocks, axis=0), blocks[3]


def _rule_kernel(q_ref, wr0_ref, br0_ref, wr1_ref, br1_ref, c_ref,
                 wl0_ref, bl0_ref, wl1_ref, bl1_ref,
                 w1_ref, b1_ref, w2_ref, emb_ref,
                 sub_ref,
                 hid1_s, rel1_s, qw1_s, score_s, wf1_s, beff1_s):
    i = pl.program_id(0)

    @pl.when(i == 0)
    def _layer0():
        q = q_ref[...]
        c = c_ref[...]
        bl0 = bl0_ref[...]
        wf0, w0k3 = _fold(wl0_ref[...])
        wf1, _ = _fold(wl1_ref[...])
        wf1_s[...] = wf1
        h2l0 = jnp.maximum(
            _STDC * jnp.sum(w0k3, axis=0, keepdims=True) + bl0, 0.0)
        beff1_s[...] = (
            jnp.dot(h2l0, wl1_ref[:D, :], preferred_element_type=F32)
            + bl1_ref[...])
        for r in range(R2):
            lo, hi = r * D, (r + 1) * D
            rel1_s[r * B:(r + 1) * B, :] = (
                jnp.dot(q, wr1_ref[:, lo:hi], preferred_element_type=F32)
                + br1_ref[:, lo:hi])
            hid1_s[r * B:(r + 1) * B, :] = c * (
                jnp.dot(q, wr0_ref[:, lo:hi], preferred_element_type=F32)
                + br0_ref[:, lo:hi])
        qw1 = (jnp.dot(q, w1_ref[D:, :], preferred_element_type=F32)
               + b1_ref[...])
        qw1_s[...] = jnp.broadcast_to(qw1[None], (R2, B, D)).reshape(R2 * B, D)
        f1 = _features(hid1_s[...])
        hid1_s[...] = jnp.maximum(
            jnp.dot(f1, wf0, preferred_element_type=F32) + bl0, 0.0)

    h1 = hid1_s[pl.ds(i * B, B), :]
    m2 = (jnp.broadcast_to(h1[None], (R2, B, D)).reshape(R2 * B, D)
          * rel1_s[...])
    f2 = _features(m2)
    hid2 = jnp.maximum(
        jnp.dot(f2, wf1_s[...], preferred_element_type=F32)
        + beff1_s[...], 0.0)
    ho = jnp.maximum(
        jnp.dot(hid2, w1_ref[:D, :], preferred_element_type=F32)
        + qw1_s[...], 0.0)
    sc = jnp.dot(ho, w2_ref[...], preferred_element_type=F32)
    score_s[i] = jnp.concatenate(
        [sc[r * B:(r + 1) * B, :] for r in range(R2)], axis=1)

    @pl.when(i == R2 - 1)
    def _finish():
        s_all = score_s[...]
        mx = jnp.max(jnp.max(s_all, axis=0), axis=1)[None, :, None]
        e = jnp.exp(s_all - mx)
        den = jnp.sum(jnp.sum(e, axis=0), axis=1)[None, :, None]
        att = e / den
        marg0 = jnp.sum(att, axis=2)
        marg1 = jnp.sum(att, axis=0)
        emb = emb_ref[...]
        sub_ref[:, 0, :] = jax.lax.dot_general(
            marg0, emb, (((0,), (0,)), ((), ())), preferred_element_type=F32)
        sub_ref[:, 1, :] = jnp.dot(marg1, emb, preferred_element_type=F32)


@functools.partial(jax.jit, static_argnames=("interpret",))
def _run(query, relation_emb, indicator, Wr0, br0, Wl0, bl0,
         Wr1, br1, Wl1, bl1, W1, b1, W2, interpret=False):
    spec = lambda shape: pl.BlockSpec(shape, lambda i: tuple(0 for _ in shape))
    return pl.pallas_call(
        _rule_kernel,
        grid=(R2,),
        in_specs=[
            spec((B, D)),
            spec((D, R2 * D)),
            spec((1, R2 * D)),
            spec((D, R2 * D)),
            spec((1, R2 * D)),
            spec((1, D)),
            spec((13 * D, D)),
            spec((1, D)),
            spec((13 * D, D)),
            spec((1, D)),
            spec((2 * D, D)),
            spec((1, D)),
            spec((D, 1)),
            spec((R2, D)),
        ],
        out_specs=spec((B, 2, D)),
        out_shape=jax.ShapeDtypeStruct((B, 2, D), F32),
        scratch_shapes=[
            pltpu.VMEM((R2 * B, D), F32),
            pltpu.VMEM((R2 * B, D), F32),
            pltpu.VMEM((R2 * B, D), F32),
            pltpu.VMEM((R2, B, R2), F32),
            pltpu.VMEM((4 * D, D), F32),
            pltpu.VMEM((1, D), F32),
        ],
        interpret=interpret,
    )(query, Wr0, br0.reshape(1, R2 * D), Wr1, br1.reshape(1, R2 * D),
      indicator, Wl0, bl0.reshape(1, D), Wl1, bl1.reshape(1, D),
      W1, b1.reshape(1, D), W2, relation_emb)


def kernel(query, relation_emb, indicator, Wr0, br0, Wl0, bl0,
           Wr1, br1, Wl1, bl1, W1, b1, W2, b2):
    subgoals = _run(query, relation_emb, indicator, Wr0, br0, Wl0, bl0,
                    Wr1, br1, Wl1, bl1, W1, b1, W2)
    masks = jnp.ones(subgoals.shape[:-1], dtype=bool)
    return (subgoals, masks)
